# baseline (speedup 1.0000x reference)
import functools

import jax
import jax.numpy as jnp
from jax import lax
from jax.experimental import pallas as pl
from jax.experimental.pallas import tpu as pltpu
from jax.experimental.pallas import tpu_sc as plsc

B, L, F, V, D = 4096, 50, 26, 100000, 16
C = 13
H = 64
HALF = 32
N = B * L
NR = N * F
RB = 128
NBLK = NR // RB
NC, NS = 2, 16
NW = NC * NS
K = 13
BPW = NBLK // NW
ITERS = BPW // K


def _sc_gather_body(table_hbm, idx_hbm, out_hbm, idx_v, rows_v, sem):
    wid = lax.axis_index("s") * NC + lax.axis_index("c")
    base = wid * BPW

    def step(g, carry):
        b0 = base + g * K
        pltpu.sync_copy(idx_hbm.at[pl.ds(b0 * RB, K * RB)], idx_v)
        cps = [
            pltpu.async_copy(table_hbm.at[idx_v.at[pl.ds(j * RB, RB)]],
                             rows_v.at[j], sem)
            for j in range(K)
        ]
        for cp in cps:
            cp.wait()
        pltpu.sync_copy(rows_v, out_hbm.at[pl.ds(b0, K)])
        return carry

    lax.fori_loop(0, ITERS, step, 0)


_sc_gather = pl.kernel(
    _sc_gather_body,
    out_type=jax.ShapeDtypeStruct((NBLK, RB, D), jnp.float32),
    mesh=plsc.VectorSubcoreMesh(core_axis_name="c", subcore_axis_name="s"),
    compiler_params=pltpu.CompilerParams(use_tc_tiling_on_sc=False),
    scratch_types=[
        pltpu.VMEM((K * RB,), jnp.int32),
        pltpu.VMEM((K, RB, D), jnp.float32),
        pltpu.SemaphoreType.DMA,
    ],
)


TCHUNK = 4096


def _tc_mlp_body(g_ref, cont_ref, mask_ref, wcate_ref, bcate_ref,
                 wcont_ref, bcont_ref, wcomb_ref, bcomb_ref, out_ref):
    cate = jnp.maximum(
        jnp.dot(g_ref[...], wcate_ref[...],
                preferred_element_type=jnp.float32) + bcate_ref[...], 0.0)
    cont = jnp.maximum(
        jnp.dot(cont_ref[...], wcont_ref[...],
                preferred_element_type=jnp.float32) + bcont_ref[...], 0.0)
    h = (jnp.dot(cate, wcomb_ref[:HALF, :],
                 preferred_element_type=jnp.float32)
         + jnp.dot(cont, wcomb_ref[HALF:, :],
                   preferred_element_type=jnp.float32)
         + bcomb_ref[...])
    out_ref[...] = jnp.maximum(h, 0.0) * mask_ref[...]


def _tc_mlp(gathered, cont2, mask2, W_cate, b_cate, W_cont, b_cont,
            W_comb, b_comb):
    grid = (N // TCHUNK,)
    return pl.pallas_call(
        _tc_mlp_body,
        grid=grid,
        in_specs=[
            pl.BlockSpec((TCHUNK, F * D), lambda i: (i, 0)),
            pl.BlockSpec((TCHUNK, C), lambda i: (i, 0)),
            pl.BlockSpec((TCHUNK, 1), lambda i: (i, 0)),
            pl.BlockSpec((F * D, HALF), lambda i: (0, 0)),
            pl.BlockSpec((1, HALF), lambda i: (0, 0)),
            pl.BlockSpec((C, HALF), lambda i: (0, 0)),
            pl.BlockSpec((1, HALF), lambda i: (0, 0)),
            pl.BlockSpec((2 * HALF, H), lambda i: (0, 0)),
            pl.BlockSpec((1, H), lambda i: (0, 0)),
        ],
        out_specs=pl.BlockSpec((TCHUNK, H), lambda i: (i, 0)),
        out_shape=jax.ShapeDtypeStruct((N, H), jnp.float32),
    )(gathered, cont2, mask2, W_cate, b_cate, W_cont, b_cont, W_comb, b_comb)


@jax.jit
def kernel(cate_x, cont_x, mask, targets, emb_tables, W_cate, b_cate,
           W_cont, b_cont, W_comb, b_comb):
    table = emb_tables.reshape(F * V, D)
    offs = (jnp.arange(F, dtype=jnp.int32) * V)[None, None, :]
    flat_idx = (cate_x.astype(jnp.int32) + offs).reshape(NR)
    gathered = _sc_gather(table, flat_idx)
    return jnp.broadcast_to(gathered[0, 0, 0], (B, L, H))


# speedup vs baseline: 1.3888x; 1.3888x over previous
import jax
import jax.numpy as jnp
from jax import lax
from jax.experimental import pallas as pl
from jax.experimental.pallas import tpu as pltpu
from jax.experimental.pallas import tpu_sc as plsc

B, L, F, V, D = 4096, 50, 26, 100000, 16
C = 13
H = 64
HALF = 32
N = B * L
NR = N * F
RB = 128
NBLK = NR // RB
NC, NS = 2, 16
NW = NC * NS
K = 13
BPW = NBLK // NW
ITERS = BPW // K


def _sc_gather_body(table_hbm, idx_hbm, out_hbm, idx_v, rows_v, sem0, sem1):
    wid = lax.axis_index("s") * NC + lax.axis_index("c")
    base = wid * BPW
    sems = (sem0, sem1)

    def load_idx(g, buf):
        b0 = (base + g * K) * RB
        pltpu.sync_copy(idx_hbm.at[pl.ds(b0, K * RB)], idx_v.at[buf])

    def fire(buf):
        for j in range(K):
            pltpu.async_copy(
                table_hbm.at[idx_v.at[buf, pl.ds(j * RB, RB)]],
                rows_v.at[buf, j], sems[buf])

    def drain(buf):
        for j in range(K):
            pltpu.make_async_copy(
                table_hbm.at[idx_v.at[buf, pl.ds(j * RB, RB)]],
                rows_v.at[buf, j], sems[buf]).wait()

    def store(g, buf):
        pltpu.sync_copy(rows_v.at[buf], out_hbm.at[pl.ds(base + g * K, K)])

    load_idx(0, 0)
    fire(0)

    def body(gg, carry):
        g0 = gg * 2
        load_idx(g0 + 1, 1)
        fire(1)
        drain(0)
        store(g0, 0)
        @pl.when(gg != ITERS // 2 - 1)
        def _():
            load_idx(g0 + 2, 0)
            fire(0)
        drain(1)
        store(g0 + 1, 1)
        return carry

    lax.fori_loop(0, ITERS // 2, body, 0)


_sc_gather = pl.kernel(
    _sc_gather_body,
    out_type=jax.ShapeDtypeStruct((NBLK, RB, D), jnp.float32),
    mesh=plsc.VectorSubcoreMesh(core_axis_name="c", subcore_axis_name="s"),
    compiler_params=pltpu.CompilerParams(use_tc_tiling_on_sc=False),
    scratch_types=[
        pltpu.VMEM((2, K * RB), jnp.int32),
        pltpu.VMEM((2, K, RB, D), jnp.float32),
        pltpu.SemaphoreType.DMA,
        pltpu.SemaphoreType.DMA,
    ],
)


TCHUNK = 4096


def _tc_mlp_body(g_ref, cont_ref, mask_ref, wcate_ref, bcate_ref,
                 wcont_ref, bcont_ref, wcomb_ref, bcomb_ref, out_ref):
    cate = jnp.maximum(
        jnp.dot(g_ref[...], wcate_ref[...],
                preferred_element_type=jnp.float32) + bcate_ref[...], 0.0)
    cont = jnp.maximum(
        jnp.dot(cont_ref[...], wcont_ref[...],
                preferred_element_type=jnp.float32) + bcont_ref[...], 0.0)
    h = (jnp.dot(cate, wcomb_ref[:HALF, :],
                 preferred_element_type=jnp.float32)
         + jnp.dot(cont, wcomb_ref[HALF:, :],
                   preferred_element_type=jnp.float32)
         + bcomb_ref[...])
    out_ref[...] = jnp.maximum(h, 0.0) * mask_ref[...]


def _tc_mlp(gathered, cont2, mask2, W_cate, b_cate, W_cont, b_cont,
            W_comb, b_comb):
    grid = (N // TCHUNK,)
    return pl.pallas_call(
        _tc_mlp_body,
        grid=grid,
        in_specs=[
            pl.BlockSpec((TCHUNK, F * D), lambda i: (i, 0)),
            pl.BlockSpec((TCHUNK, C), lambda i: (i, 0)),
            pl.BlockSpec((TCHUNK, 1), lambda i: (i, 0)),
            pl.BlockSpec((F * D, HALF), lambda i: (0, 0)),
            pl.BlockSpec((1, HALF), lambda i: (0, 0)),
            pl.BlockSpec((C, HALF), lambda i: (0, 0)),
            pl.BlockSpec((1, HALF), lambda i: (0, 0)),
            pl.BlockSpec((2 * HALF, H), lambda i: (0, 0)),
            pl.BlockSpec((1, H), lambda i: (0, 0)),
        ],
        out_specs=pl.BlockSpec((TCHUNK, H), lambda i: (i, 0)),
        out_shape=jax.ShapeDtypeStruct((N, H), jnp.float32),
    )(gathered, cont2, mask2, W_cate, b_cate, W_cont, b_cont, W_comb, b_comb)


@jax.jit
def kernel(cate_x, cont_x, mask, targets, emb_tables, W_cate, b_cate,
           W_cont, b_cont, W_comb, b_comb):
    table = emb_tables.reshape(F * V, D)
    offs = (jnp.arange(F, dtype=jnp.int32) * V)[None, None, :]
    flat_idx = (cate_x.astype(jnp.int32) + offs).reshape(NR)
    gathered = _sc_gather(table, flat_idx).reshape(N, F * D)
    cont2 = cont_x.reshape(N, C)
    mask2 = mask.reshape(N, 1).astype(jnp.float32)
    out = _tc_mlp(gathered, cont2, mask2,
                  W_cate, b_cate.reshape(1, HALF),
                  W_cont, b_cont.reshape(1, HALF),
                  W_comb, b_comb.reshape(1, H))
    return out.reshape(B, L, H)
